# Initial kernel scaffold; baseline (speedup 1.0000x reference)
import jax
import jax.numpy as jnp
from jax import lax
from jax.experimental import pallas as pl
from jax.experimental.pallas import tpu as pltpu
from jax.experimental.pallas import tpu_sc as plsc

N_ROWS = 16384
N_COLS = 50
D = 32
B_TOTAL = N_ROWS * N_COLS

_info = plsc.get_sparse_core_info()
NC = _info.num_cores
NS = _info.num_subcores
NW = NC * NS
B_PER_W = B_TOTAL // NW

CHUNK = 1280
N_CHUNKS = B_PER_W // CHUNK


def _gather_kernel(idx_hbm, table_hbm, out_hbm, idx_v, rows_v, sem):
    wid = lax.axis_index("s") * NC + lax.axis_index("c")
    base = wid * B_PER_W

    def chunk_body(i, carry):
        off = base + i * CHUNK
        pltpu.sync_copy(idx_hbm.at[pl.ds(off, CHUNK)], idx_v)
        pltpu.async_copy(table_hbm.at[idx_v], rows_v, sem).wait()
        pltpu.sync_copy(rows_v, out_hbm.at[pl.ds(off, CHUNK)])
        return carry

    lax.fori_loop(0, N_CHUNKS, chunk_body, 0)


@jax.jit
def _gather(idx, weight):
    mesh = plsc.VectorSubcoreMesh(core_axis_name="c", subcore_axis_name="s")
    return pl.kernel(
        _gather_kernel,
        out_type=jax.ShapeDtypeStruct((B_TOTAL, D), jnp.float32),
        mesh=mesh,
        scratch_types=[
            pltpu.VMEM((CHUNK,), jnp.int32),
            pltpu.VMEM((CHUNK, D), jnp.float32),
            pltpu.SemaphoreType.DMA,
        ],
    )(idx, weight)


def kernel(states, weight):
    idx = states.reshape(-1).astype(jnp.int32)
    out = _gather(idx, weight)
    return out.reshape(N_ROWS, N_COLS, D)


# speedup vs baseline: 1.0991x; 1.0991x over previous
import jax
import jax.numpy as jnp
from jax import lax
from jax.experimental import pallas as pl
from jax.experimental.pallas import tpu as pltpu
from jax.experimental.pallas import tpu_sc as plsc

N_ROWS = 16384
N_COLS = 50
D = 32
B_TOTAL = N_ROWS * N_COLS

_info = plsc.get_sparse_core_info()
NC = _info.num_cores
NS = _info.num_subcores
NW = NC * NS
B_PER_W = B_TOTAL // NW

CHUNK = 1280
N_CHUNKS = B_PER_W // CHUNK


def _gather_kernel(idx_hbm, table_hbm, out_hbm, idx_v, rows_v, sem):
    wid = lax.axis_index("s") * NC + lax.axis_index("c")
    base = wid * B_PER_W

    def chunk_body(i, carry):
        off = base + i * CHUNK
        pltpu.sync_copy(idx_hbm.at[pl.ds(off, CHUNK)], idx_v)
        pltpu.async_copy(table_hbm.at[idx_v], rows_v, sem).wait()
        pltpu.sync_copy(rows_v, out_hbm.at[pl.ds(off, CHUNK)])
        return carry

    lax.fori_loop(0, N_CHUNKS, chunk_body, 0)


@jax.jit
def _gather(idx, weight):
    mesh = plsc.VectorSubcoreMesh(core_axis_name="c", subcore_axis_name="s")
    return pl.kernel(
        _gather_kernel,
        out_type=jax.ShapeDtypeStruct((B_TOTAL, D), jnp.float32),
        mesh=mesh,
        scratch_types=[
            pltpu.VMEM((CHUNK,), jnp.int32),
            pltpu.VMEM((CHUNK, D), jnp.float32),
            pltpu.SemaphoreType.DMA,
        ],
        compiler_params=pltpu.CompilerParams(use_tc_tiling_on_sc=False),
    )(idx, weight)


def kernel(states, weight):
    idx = states.reshape(-1).astype(jnp.int32)
    out = _gather(idx, weight)
    return out.reshape(N_ROWS, N_COLS, D)


# speedup vs baseline: 1.1084x; 1.0085x over previous
import jax
import jax.numpy as jnp
from jax import lax
from jax.experimental import pallas as pl
from jax.experimental.pallas import tpu as pltpu
from jax.experimental.pallas import tpu_sc as plsc

N_ROWS = 16384
N_COLS = 50
D = 32
B_TOTAL = N_ROWS * N_COLS

_info = plsc.get_sparse_core_info()
NC = _info.num_cores
NS = _info.num_subcores
NW = NC * NS
B_PER_W = B_TOTAL // NW

CHUNK = 1600
N_CHUNKS = B_PER_W // CHUNK
NBUF = 2


def _gather_kernel(idx_hbm, table_hbm, out_hbm, idx_v, rows_v,
                   sem_i, sem_g, sem_o):
    wid = lax.axis_index("s") * NC + lax.axis_index("c")
    base = wid * B_PER_W

    def idx_copy(i, b):
        return pltpu.async_copy(
            idx_hbm.at[pl.ds(base + i * CHUNK, CHUNK)], idx_v.at[b], sem_i)

    h_idx = {}
    for b in range(NBUF):
        h_idx[b] = idx_copy(b, b)

    h_out = {}
    for i in range(N_CHUNKS):
        b = i % NBUF
        if i >= NBUF:
            h_out[i - NBUF].wait()
        h_idx[i].wait()
        g = pltpu.async_copy(table_hbm.at[idx_v.at[b]], rows_v.at[b], sem_g)
        g.wait()
        if i + NBUF < N_CHUNKS:
            h_idx[i + NBUF] = idx_copy(i + NBUF, b)
        h_out[i] = pltpu.async_copy(
            rows_v.at[b], out_hbm.at[pl.ds(base + i * CHUNK, CHUNK)], sem_o)

    for i in range(N_CHUNKS - NBUF, N_CHUNKS):
        h_out[i].wait()


@jax.jit
def _gather(idx, weight):
    mesh = plsc.VectorSubcoreMesh(core_axis_name="c", subcore_axis_name="s")
    return pl.kernel(
        _gather_kernel,
        out_type=jax.ShapeDtypeStruct((B_TOTAL, D), jnp.float32),
        mesh=mesh,
        scratch_types=[
            pltpu.VMEM((NBUF, CHUNK), jnp.int32),
            pltpu.VMEM((NBUF, CHUNK, D), jnp.float32),
            pltpu.SemaphoreType.DMA,
            pltpu.SemaphoreType.DMA,
            pltpu.SemaphoreType.DMA,
        ],
        compiler_params=pltpu.CompilerParams(use_tc_tiling_on_sc=False),
    )(idx, weight)


def kernel(states, weight):
    idx = states.reshape(-1).astype(jnp.int32)
    out = _gather(idx, weight)
    return out.reshape(N_ROWS, N_COLS, D)


# speedup vs baseline: 1.4113x; 1.2732x over previous
import jax
import jax.numpy as jnp
from jax import lax
from jax.experimental import pallas as pl
from jax.experimental.pallas import tpu as pltpu
from jax.experimental.pallas import tpu_sc as plsc

N_ROWS = 16384
N_COLS = 50
D = 32
B_TOTAL = N_ROWS * N_COLS
UNIT = 128
N_UNITS = B_TOTAL // UNIT

_info = plsc.get_sparse_core_info()
NC = _info.num_cores
NS = _info.num_subcores
NW = NC * NS
U_PER_W = N_UNITS // NW
NBUF = 2


def _transpose_unit(rows_ref, tile_ref):
    row_iota = lax.iota(jnp.int32, 16)
    for c in range(D):
        col = jnp.full((16,), c, jnp.int32)
        for k in range(UNIT // 16):
            vec = plsc.load_gather(rows_ref, [row_iota + 16 * k, col])
            tile_ref[c, pl.ds(16 * k, 16)] = vec


def _gather_kernel(idx_hbm, table_hbm, out_hbm, idx_v, rows_v, tile_v,
                   sem_g0, sem_g1, sem_o0, sem_o1):
    wid = lax.axis_index("s") * NC + lax.axis_index("c")
    u0 = wid * U_PER_W
    sem_g = (sem_g0, sem_g1)
    sem_o = (sem_o0, sem_o1)

    pltpu.sync_copy(idx_hbm.at[pl.ds(UNIT * u0, UNIT * U_PER_W)], idx_v)

    def gather_unit(local_u, b):
        return pltpu.async_copy(
            table_hbm.at[idx_v.at[pl.ds(UNIT * local_u, UNIT)]],
            rows_v.at[b], sem_g[b])

    for b in range(NBUF):
        gather_unit(b, b)

    def body(g, carry):
        for b in range(NBUF):
            local_u = NBUF * g + b
            u = u0 + local_u
            pltpu.make_async_copy(
                table_hbm.at[pl.ds(0, UNIT), :], rows_v.at[b], sem_g[b]
            ).wait()
            @pl.when(g > 0)
            def _():
                pltpu.make_async_copy(
                    tile_v.at[b], out_hbm.at[0, :, pl.ds(0, UNIT)], sem_o[b]
                ).wait()
            _transpose_unit(rows_v.at[b], tile_v.at[b])
            @pl.when(local_u + NBUF < U_PER_W)
            def _():
                gather_unit(local_u + NBUF, b)
            s = u // UNIT
            ti = u - s * UNIT
            pltpu.async_copy(
                tile_v.at[b], out_hbm.at[s, :, pl.ds(UNIT * ti, UNIT)],
                sem_o[b])
        return carry

    lax.fori_loop(0, U_PER_W // NBUF, body, 0)

    for b in range(NBUF):
        pltpu.make_async_copy(
            tile_v.at[b], out_hbm.at[0, :, pl.ds(0, UNIT)], sem_o[b]
        ).wait()


@jax.jit
def _gather(idx, weight):
    mesh = plsc.VectorSubcoreMesh(core_axis_name="c", subcore_axis_name="s")
    return pl.kernel(
        _gather_kernel,
        out_type=jax.ShapeDtypeStruct((N_COLS, D, N_ROWS), jnp.float32),
        mesh=mesh,
        scratch_types=[
            pltpu.VMEM((UNIT * U_PER_W,), jnp.int32),
            pltpu.VMEM((NBUF, UNIT, D), jnp.float32),
            pltpu.VMEM((NBUF, D, UNIT), jnp.float32),
            pltpu.SemaphoreType.DMA,
            pltpu.SemaphoreType.DMA,
            pltpu.SemaphoreType.DMA,
            pltpu.SemaphoreType.DMA,
        ],
        compiler_params=pltpu.CompilerParams(
            use_tc_tiling_on_sc=False, needs_layout_passes=False),
    )(idx, weight)


def kernel(states, weight):
    idx_sm = jnp.transpose(states).reshape(-1).astype(jnp.int32)
    t4 = _gather(idx_sm, weight)
    return jnp.transpose(t4, (2, 0, 1))
